# baseline (speedup 1.0000x reference)
import jax
import jax.numpy as jnp
import numpy as np
from jax import lax
from jax.experimental import pallas as pl
from jax.experimental.pallas import tpu as pltpu

L = 2048
DM = 1024
H = 16
DH = 64
DFF = 2048
U = 64
SK = 128
SCALE = 1.0 / np.sqrt(DH)
RT = 256
F32 = jnp.float32
BF = jnp.bfloat16


def _sel_matrix():
    sidx = jax.random.permutation(jax.random.key(42), L)[:SK]
    return jax.nn.one_hot(sidx, L, dtype=F32)


def _dot(a, b, dims=None):
    if dims is None:
        dims = (((1,), (0,)), ((), ()))
    return lax.dot_general(a, b, dims, precision=lax.Precision.DEFAULT,
                           preferred_element_type=F32)


def _qkv_body(x_ref, wq_ref, wk_ref, wv_bf_ref, bq_ref, bk_ref, bv_ref,
              q_ref, k_ref, v_ref):
    xx = x_ref[...]
    q_ref[...] = _dot(xx, wq_ref[...]) + bq_ref[...]
    k_ref[...] = _dot(xx, wk_ref[...]) + bk_ref[...]
    v_ref[...] = (_dot(xx.astype(BF), wv_bf_ref[...])
                  + bv_ref[...]).astype(BF)


def _m_body(q_ref, k_ref, sel_ref, m_ref):
    ks2 = _dot(sel_ref[...], k_ref[...])
    for hh in range(2):
        qh = q_ref[:, hh * DH:(hh + 1) * DH]
        ksh = ks2[:, hh * DH:(hh + 1) * DH]
        ssT = _dot(ksh, qh, (((1,), (1,)), ((), ()))) * SCALE
        m_ref[hh, 0, :] = (jnp.max(ssT, axis=0)
                           - jnp.mean(ssT, axis=0))


def _sel_body(mr_ref, mc_ref, pt_ref):
    uio = lax.broadcasted_iota(jnp.int32, (RT, U), 1).astype(F32)
    ones = jnp.ones((L, 1), BF)

    def chunk_body(c, carry):
        base = c * RT
        colid = lax.broadcasted_iota(jnp.int32, (RT, L), 1)
        rowid = lax.broadcasted_iota(jnp.int32, (RT, L), 0) + base
        lt = colid < rowid
        for h in range(H):
            mr = mr_ref[h:h + 1, :]
            mc = mc_ref[pl.ds(base, RT), h:h + 1]
            cmp = (mr > mc) | ((mr == mc) & lt)
            rank = _dot(jnp.where(cmp, 1.0, 0.0).astype(BF), ones)
            pt_ref[h, pl.ds(base, RT), :] = (rank == uio).astype(BF)
        return carry

    lax.fori_loop(0, L // RT, chunk_body, 0)


def _attn_body(q_ref, k_ref, v_ref, pt_ref, ctx_ref):
    for hh in range(2):
        cs, ce = hh * DH, (hh + 1) * DH
        pt = pt_ref[hh].astype(F32)
        qh = q_ref[:, cs:ce]
        kh = k_ref[:, cs:ce]
        vh_bf = v_ref[:, cs:ce]
        qtop = _dot(pt, qh, (((0,), (0,)), ((), ())))
        s = _dot(qtop, kh, (((1,), (1,)), ((), ()))) * SCALE
        smax = jnp.max(s, axis=1, keepdims=True)
        e = jnp.exp(s - smax)
        a = e / jnp.sum(e, axis=1, keepdims=True)
        otop = _dot(a.astype(BF), vh_bf)
        vmean = jnp.mean(vh_bf.astype(F32), axis=0, keepdims=True)
        notsel = 1.0 - jnp.sum(pt, axis=1, keepdims=True)
        ctx_ref[:, cs:ce] = _dot(pt, otop) + notsel * vmean


def _ln(y, g, b):
    mu = jnp.mean(y, axis=1, keepdims=True)
    var = jnp.mean((y - mu) * (y - mu), axis=1, keepdims=True)
    return (y - mu) / jnp.sqrt(var + 1e-6) * g + b


def _ffn_body(x_ref, ctx_ref, wo_bf_ref, bo_ref, g1_ref, b1n_ref,
              w1_bf_ref, b1_ref, w2_bf_ref, b2_ref, g2_ref, b2n_ref,
              out_ref):
    attn = _dot(ctx_ref[...].astype(BF), wo_bf_ref[...]) + bo_ref[...]
    h1 = _ln(x_ref[...] + attn, g1_ref[...], b1n_ref[...])
    t = jnp.maximum(_dot(h1.astype(BF), w1_bf_ref[...]) + b1_ref[...], 0.0)
    f = _dot(t.astype(BF), w2_bf_ref[...]) + b2_ref[...]
    out_ref[...] = _ln(h1 + f, g2_ref[...], b2n_ref[...])


def kernel(x, Wq, bq, Wk, bk, Wv, bv, Wo, bo, ln1_g, ln1_b,
           W1, b1, W2, b2, ln2_g, ln2_b):
    x2 = x.reshape(L, DM)
    row = lambda t: t.reshape(1, -1)
    nrt = L // RT

    q, k, v = pl.pallas_call(
        _qkv_body,
        grid=(nrt,),
        in_specs=[
            pl.BlockSpec((RT, DM), lambda i: (i, 0)),
            pl.BlockSpec((DM, DM), lambda i: (0, 0)),
            pl.BlockSpec((DM, DM), lambda i: (0, 0)),
            pl.BlockSpec((DM, DM), lambda i: (0, 0)),
            pl.BlockSpec((1, DM), lambda i: (0, 0)),
            pl.BlockSpec((1, DM), lambda i: (0, 0)),
            pl.BlockSpec((1, DM), lambda i: (0, 0)),
        ],
        out_specs=[
            pl.BlockSpec((RT, DM), lambda i: (i, 0)),
            pl.BlockSpec((RT, DM), lambda i: (i, 0)),
            pl.BlockSpec((RT, DM), lambda i: (i, 0)),
        ],
        out_shape=[
            jax.ShapeDtypeStruct((L, DM), F32),
            jax.ShapeDtypeStruct((L, DM), F32),
            jax.ShapeDtypeStruct((L, DM), BF),
        ],
    )(x2, Wq, Wk, Wv.astype(BF), row(bq), row(bk), row(bv))

    m3 = pl.pallas_call(
        _m_body,
        grid=(H // 2,),
        in_specs=[
            pl.BlockSpec((L, 2 * DH), lambda g: (0, g)),
            pl.BlockSpec((L, 2 * DH), lambda g: (0, g)),
            pl.BlockSpec((SK, L), lambda g: (0, 0)),
        ],
        out_specs=pl.BlockSpec((2, 1, L), lambda g: (g, 0, 0)),
        out_shape=jax.ShapeDtypeStruct((H, 1, L), F32),
    )(q, k, _sel_matrix())

    m_all = m3.reshape(H, L)
    m_t = m_all.T

    pt_all = pl.pallas_call(
        _sel_body,
        out_shape=jax.ShapeDtypeStruct((H, L, U), BF),
    )(m_all, m_t)

    ctx = pl.pallas_call(
        _attn_body,
        grid=(H // 2,),
        in_specs=[
            pl.BlockSpec((L, 2 * DH), lambda g: (0, g)),
            pl.BlockSpec((L, 2 * DH), lambda g: (0, g)),
            pl.BlockSpec((L, 2 * DH), lambda g: (0, g)),
            pl.BlockSpec((2, L, U), lambda g: (g, 0, 0)),
        ],
        out_specs=pl.BlockSpec((L, 2 * DH), lambda g: (0, g)),
        out_shape=jax.ShapeDtypeStruct((L, DM), F32),
    )(q, k, v, pt_all)

    out = pl.pallas_call(
        _ffn_body,
        grid=(nrt,),
        in_specs=[
            pl.BlockSpec((RT, DM), lambda i: (i, 0)),
            pl.BlockSpec((RT, DM), lambda i: (i, 0)),
            pl.BlockSpec((DM, DM), lambda i: (0, 0)),
            pl.BlockSpec((1, DM), lambda i: (0, 0)),
            pl.BlockSpec((1, DM), lambda i: (0, 0)),
            pl.BlockSpec((1, DM), lambda i: (0, 0)),
            pl.BlockSpec((DM, DFF), lambda i: (0, 0)),
            pl.BlockSpec((1, DFF), lambda i: (0, 0)),
            pl.BlockSpec((DFF, DM), lambda i: (0, 0)),
            pl.BlockSpec((1, DM), lambda i: (0, 0)),
            pl.BlockSpec((1, DM), lambda i: (0, 0)),
            pl.BlockSpec((1, DM), lambda i: (0, 0)),
        ],
        out_specs=pl.BlockSpec((RT, DM), lambda i: (i, 0)),
        out_shape=jax.ShapeDtypeStruct((L, DM), F32),
    )(x2, ctx, Wo.astype(BF), row(bo), row(ln1_g), row(ln1_b),
      W1.astype(BF), row(b1), W2.astype(BF), row(b2), row(ln2_g),
      row(ln2_b))

    return out.reshape(1, L, DM)


# speedup vs baseline: 1.0317x; 1.0317x over previous
import jax
import jax.numpy as jnp
import numpy as np
from jax import lax
from jax.experimental import pallas as pl
from jax.experimental.pallas import tpu as pltpu

L = 2048
DM = 1024
H = 16
DH = 64
DFF = 2048
U = 64
SK = 128
SCALE = 1.0 / np.sqrt(DH)
RT = 256
F32 = jnp.float32
BF = jnp.bfloat16


def _sel_matrix():
    sidx = jax.random.permutation(jax.random.key(42), L)[:SK]
    return jax.nn.one_hot(sidx, L, dtype=F32)


def _dot(a, b, dims=None):
    if dims is None:
        dims = (((1,), (0,)), ((), ()))
    return lax.dot_general(a, b, dims, precision=lax.Precision.DEFAULT,
                           preferred_element_type=F32)


def _qkv_body(x_ref, wq_ref, wk_ref, wv_bf_ref, bq_ref, bk_ref, bv_ref,
              q_ref, k_ref, v_ref):
    xx = x_ref[...]
    q_ref[...] = _dot(xx, wq_ref[...]) + bq_ref[...]
    k_ref[...] = _dot(xx, wk_ref[...]) + bk_ref[...]
    v_ref[...] = (_dot(xx.astype(BF), wv_bf_ref[...])
                  + bv_ref[...]).astype(BF)


def _m_body(q_ref, k_ref, sel_ref, m_ref):
    ks = _dot(sel_ref[...], k_ref[...])
    for h in range(H):
        qh = q_ref[:, h * DH:(h + 1) * DH]
        ksh = ks[:, h * DH:(h + 1) * DH]
        ssT = _dot(ksh, qh, (((1,), (1,)), ((), ()))) * SCALE
        m_ref[h, 0, :] = (jnp.max(ssT, axis=0)
                          - jnp.mean(ssT, axis=0))


def _sel_body(mr_ref, mc_ref, pt_ref):
    uio = lax.broadcasted_iota(jnp.int32, (RT, U), 1)

    def chunk_body(c, carry):
        base = c * RT
        colid = lax.broadcasted_iota(jnp.int32, (RT, L), 1)
        rowid = lax.broadcasted_iota(jnp.int32, (RT, L), 0) + base
        lt = colid < rowid
        for h in range(H):
            mr = mr_ref[h:h + 1, :]
            mc = mc_ref[pl.ds(base, RT), h:h + 1]
            cmp = (mr > mc) | ((mr == mc) & lt)
            rank = jnp.sum(cmp.astype(jnp.int32), axis=1, keepdims=True)
            pt_ref[h, pl.ds(base, RT), :] = (rank == uio).astype(BF)
        return carry

    lax.fori_loop(0, L // RT, chunk_body, 0)


def _attn_body(q_ref, k_ref, v_ref, pt_ref, ctx_ref):
    for hh in range(2):
        cs, ce = hh * DH, (hh + 1) * DH
        pt = pt_ref[hh].astype(F32)
        qh = q_ref[:, cs:ce]
        kh = k_ref[:, cs:ce]
        vh_bf = v_ref[:, cs:ce]
        qtop = _dot(pt, qh, (((0,), (0,)), ((), ())))
        s = _dot(qtop, kh, (((1,), (1,)), ((), ()))) * SCALE
        smax = jnp.max(s, axis=1, keepdims=True)
        e = jnp.exp(s - smax)
        a = e / jnp.sum(e, axis=1, keepdims=True)
        otop = _dot(a.astype(BF), vh_bf)
        vmean = jnp.mean(vh_bf.astype(F32), axis=0, keepdims=True)
        notsel = 1.0 - jnp.sum(pt, axis=1, keepdims=True)
        ctx_ref[:, cs:ce] = _dot(pt, otop) + notsel * vmean


def _ln(y, g, b):
    mu = jnp.mean(y, axis=1, keepdims=True)
    var = jnp.mean((y - mu) * (y - mu), axis=1, keepdims=True)
    return (y - mu) / jnp.sqrt(var + 1e-6) * g + b


def _ffn_body(x_ref, ctx_ref, wo_bf_ref, bo_ref, g1_ref, b1n_ref,
              w1_bf_ref, b1_ref, w2_bf_ref, b2_ref, g2_ref, b2n_ref,
              out_ref):
    attn = _dot(ctx_ref[...].astype(BF), wo_bf_ref[...]) + bo_ref[...]
    h1 = _ln(x_ref[...] + attn, g1_ref[...], b1n_ref[...])
    t = jnp.maximum(_dot(h1.astype(BF), w1_bf_ref[...]) + b1_ref[...], 0.0)
    f = _dot(t.astype(BF), w2_bf_ref[...]) + b2_ref[...]
    out_ref[...] = _ln(h1 + f, g2_ref[...], b2n_ref[...])


def kernel(x, Wq, bq, Wk, bk, Wv, bv, Wo, bo, ln1_g, ln1_b,
           W1, b1, W2, b2, ln2_g, ln2_b):
    x2 = x.reshape(L, DM)
    row = lambda t: t.reshape(1, -1)
    nrt = L // RT

    q, k, v = pl.pallas_call(
        _qkv_body,
        grid=(nrt,),
        in_specs=[
            pl.BlockSpec((RT, DM), lambda i: (i, 0)),
            pl.BlockSpec((DM, DM), lambda i: (0, 0)),
            pl.BlockSpec((DM, DM), lambda i: (0, 0)),
            pl.BlockSpec((DM, DM), lambda i: (0, 0)),
            pl.BlockSpec((1, DM), lambda i: (0, 0)),
            pl.BlockSpec((1, DM), lambda i: (0, 0)),
            pl.BlockSpec((1, DM), lambda i: (0, 0)),
        ],
        out_specs=[
            pl.BlockSpec((RT, DM), lambda i: (i, 0)),
            pl.BlockSpec((RT, DM), lambda i: (i, 0)),
            pl.BlockSpec((RT, DM), lambda i: (i, 0)),
        ],
        out_shape=[
            jax.ShapeDtypeStruct((L, DM), F32),
            jax.ShapeDtypeStruct((L, DM), F32),
            jax.ShapeDtypeStruct((L, DM), BF),
        ],
    )(x2, Wq, Wk, Wv.astype(BF), row(bq), row(bk), row(bv))

    m3 = pl.pallas_call(
        _m_body,
        out_shape=jax.ShapeDtypeStruct((H, 1, L), F32),
    )(q, k, _sel_matrix())

    m_all = m3.reshape(H, L)
    m_t = m_all.T

    pt_all = pl.pallas_call(
        _sel_body,
        out_shape=jax.ShapeDtypeStruct((H, L, U), BF),
    )(m_all, m_t)

    ctx = pl.pallas_call(
        _attn_body,
        grid=(H // 2,),
        in_specs=[
            pl.BlockSpec((L, 2 * DH), lambda g: (0, g)),
            pl.BlockSpec((L, 2 * DH), lambda g: (0, g)),
            pl.BlockSpec((L, 2 * DH), lambda g: (0, g)),
            pl.BlockSpec((2, L, U), lambda g: (g, 0, 0)),
        ],
        out_specs=pl.BlockSpec((L, 2 * DH), lambda g: (0, g)),
        out_shape=jax.ShapeDtypeStruct((L, DM), F32),
    )(q, k, v, pt_all)

    out = pl.pallas_call(
        _ffn_body,
        grid=(nrt,),
        in_specs=[
            pl.BlockSpec((RT, DM), lambda i: (i, 0)),
            pl.BlockSpec((RT, DM), lambda i: (i, 0)),
            pl.BlockSpec((DM, DM), lambda i: (0, 0)),
            pl.BlockSpec((1, DM), lambda i: (0, 0)),
            pl.BlockSpec((1, DM), lambda i: (0, 0)),
            pl.BlockSpec((1, DM), lambda i: (0, 0)),
            pl.BlockSpec((DM, DFF), lambda i: (0, 0)),
            pl.BlockSpec((1, DFF), lambda i: (0, 0)),
            pl.BlockSpec((DFF, DM), lambda i: (0, 0)),
            pl.BlockSpec((1, DM), lambda i: (0, 0)),
            pl.BlockSpec((1, DM), lambda i: (0, 0)),
            pl.BlockSpec((1, DM), lambda i: (0, 0)),
        ],
        out_specs=pl.BlockSpec((RT, DM), lambda i: (i, 0)),
        out_shape=jax.ShapeDtypeStruct((L, DM), F32),
    )(x2, ctx, Wo.astype(BF), row(bo), row(ln1_g), row(ln1_b),
      W1.astype(BF), row(b1), W2.astype(BF), row(b2), row(ln2_g),
      row(ln2_b))

    return out.reshape(1, L, DM)


# speedup vs baseline: 1.0489x; 1.0167x over previous
import jax
import jax.numpy as jnp
import numpy as np
from jax import lax
from jax.experimental import pallas as pl
from jax.experimental.pallas import tpu as pltpu

L = 2048
DM = 1024
H = 16
DH = 64
DFF = 2048
U = 64
SK = 128
SCALE = 1.0 / np.sqrt(DH)
RT = 256
F32 = jnp.float32
BF = jnp.bfloat16


def _sel_matrix():
    sidx = jax.random.permutation(jax.random.key(42), L)[:SK]
    return jax.nn.one_hot(sidx, L, dtype=F32)


def _dot(a, b, dims=None):
    if dims is None:
        dims = (((1,), (0,)), ((), ()))
    return lax.dot_general(a, b, dims, precision=lax.Precision.DEFAULT,
                           preferred_element_type=F32)


def _qkv_body(x_ref, wq_ref, wk_ref, wv_bf_ref, bq_ref, bk_ref, bv_ref,
              q_ref, k_ref, v_ref):
    xx = x_ref[...]
    q_ref[...] = (_dot(xx, wq_ref[...]) + bq_ref[...]).astype(BF)
    k_ref[...] = (_dot(xx, wk_ref[...]) + bk_ref[...]).astype(BF)
    v_ref[...] = (_dot(xx.astype(BF), wv_bf_ref[...])
                  + bv_ref[...]).astype(BF)


def _m_body(q_ref, k_ref, sel_ref, m_ref):
    ks = _dot(sel_ref[...].astype(BF), k_ref[...])
    for h in range(H):
        qh = q_ref[:, h * DH:(h + 1) * DH]
        ksh = ks[:, h * DH:(h + 1) * DH]
        ssT = _dot(ksh.astype(BF), qh,
                   (((1,), (1,)), ((), ()))) * SCALE
        m_ref[h, 0, :] = (jnp.max(ssT, axis=0)
                          - jnp.mean(ssT, axis=0))


def _sel_body(mr_ref, mc_ref, pt_ref):
    uio = lax.broadcasted_iota(jnp.int32, (RT, U), 1)

    def chunk_body(c, carry):
        base = c * RT
        colid = lax.broadcasted_iota(jnp.int32, (RT, L), 1)
        rowid = lax.broadcasted_iota(jnp.int32, (RT, L), 0) + base
        lt = colid < rowid
        for h in range(H):
            mr = mr_ref[h:h + 1, :]
            mc = mc_ref[pl.ds(base, RT), h:h + 1]
            cmp = (mr > mc) | ((mr == mc) & lt)
            rank = jnp.sum(cmp.astype(jnp.int32), axis=1, keepdims=True)
            pt_ref[h, pl.ds(base, RT), :] = (rank == uio).astype(BF)
        return carry

    lax.fori_loop(0, L // RT, chunk_body, 0)


def _attn_body(q_ref, k_ref, v_ref, pt_ref, ctx_ref):
    for hh in range(2):
        cs, ce = hh * DH, (hh + 1) * DH
        pt = pt_ref[hh]
        qh = q_ref[:, cs:ce]
        kh = k_ref[:, cs:ce]
        vh_bf = v_ref[:, cs:ce]
        qtop = _dot(pt, qh, (((0,), (0,)), ((), ())))
        s = _dot(qtop.astype(BF), kh,
                 (((1,), (1,)), ((), ()))) * SCALE
        smax = jnp.max(s, axis=1, keepdims=True)
        e = jnp.exp(s - smax)
        a = e / jnp.sum(e, axis=1, keepdims=True)
        otop = _dot(a.astype(BF), vh_bf)
        vmean = jnp.mean(vh_bf.astype(F32), axis=0, keepdims=True)
        notsel = 1.0 - jnp.sum(pt.astype(F32), axis=1, keepdims=True)
        ctx_ref[:, cs:ce] = _dot(pt, otop.astype(BF)) + notsel * vmean


def _ln(y, g, b):
    mu = jnp.mean(y, axis=1, keepdims=True)
    var = jnp.mean((y - mu) * (y - mu), axis=1, keepdims=True)
    return (y - mu) / jnp.sqrt(var + 1e-6) * g + b


def _ffn_body(x_ref, ctx_ref, wo_bf_ref, bo_ref, g1_ref, b1n_ref,
              w1_bf_ref, b1_ref, w2_bf_ref, b2_ref, g2_ref, b2n_ref,
              out_ref):
    attn = _dot(ctx_ref[...].astype(BF), wo_bf_ref[...]) + bo_ref[...]
    h1 = _ln(x_ref[...] + attn, g1_ref[...], b1n_ref[...])
    t = jnp.maximum(_dot(h1.astype(BF), w1_bf_ref[...]) + b1_ref[...], 0.0)
    f = _dot(t.astype(BF), w2_bf_ref[...]) + b2_ref[...]
    out_ref[...] = _ln(h1 + f, g2_ref[...], b2n_ref[...])


def kernel(x, Wq, bq, Wk, bk, Wv, bv, Wo, bo, ln1_g, ln1_b,
           W1, b1, W2, b2, ln2_g, ln2_b):
    x2 = x.reshape(L, DM)
    row = lambda t: t.reshape(1, -1)
    nrt = L // RT

    q, k, v = pl.pallas_call(
        _qkv_body,
        grid=(nrt,),
        in_specs=[
            pl.BlockSpec((RT, DM), lambda i: (i, 0)),
            pl.BlockSpec((DM, DM), lambda i: (0, 0)),
            pl.BlockSpec((DM, DM), lambda i: (0, 0)),
            pl.BlockSpec((DM, DM), lambda i: (0, 0)),
            pl.BlockSpec((1, DM), lambda i: (0, 0)),
            pl.BlockSpec((1, DM), lambda i: (0, 0)),
            pl.BlockSpec((1, DM), lambda i: (0, 0)),
        ],
        out_specs=[
            pl.BlockSpec((RT, DM), lambda i: (i, 0)),
            pl.BlockSpec((RT, DM), lambda i: (i, 0)),
            pl.BlockSpec((RT, DM), lambda i: (i, 0)),
        ],
        out_shape=[
            jax.ShapeDtypeStruct((L, DM), BF),
            jax.ShapeDtypeStruct((L, DM), BF),
            jax.ShapeDtypeStruct((L, DM), BF),
        ],
    )(x2, Wq, Wk, Wv.astype(BF), row(bq), row(bk), row(bv))

    m3 = pl.pallas_call(
        _m_body,
        out_shape=jax.ShapeDtypeStruct((H, 1, L), F32),
    )(q, k, _sel_matrix())

    m_all = m3.reshape(H, L)
    m_t = m_all.T

    pt_all = pl.pallas_call(
        _sel_body,
        out_shape=jax.ShapeDtypeStruct((H, L, U), BF),
    )(m_all, m_t)

    ctx = pl.pallas_call(
        _attn_body,
        grid=(H // 2,),
        in_specs=[
            pl.BlockSpec((L, 2 * DH), lambda g: (0, g)),
            pl.BlockSpec((L, 2 * DH), lambda g: (0, g)),
            pl.BlockSpec((L, 2 * DH), lambda g: (0, g)),
            pl.BlockSpec((2, L, U), lambda g: (g, 0, 0)),
        ],
        out_specs=pl.BlockSpec((L, 2 * DH), lambda g: (0, g)),
        out_shape=jax.ShapeDtypeStruct((L, DM), F32),
    )(q, k, v, pt_all)

    out = pl.pallas_call(
        _ffn_body,
        grid=(nrt,),
        in_specs=[
            pl.BlockSpec((RT, DM), lambda i: (i, 0)),
            pl.BlockSpec((RT, DM), lambda i: (i, 0)),
            pl.BlockSpec((DM, DM), lambda i: (0, 0)),
            pl.BlockSpec((1, DM), lambda i: (0, 0)),
            pl.BlockSpec((1, DM), lambda i: (0, 0)),
            pl.BlockSpec((1, DM), lambda i: (0, 0)),
            pl.BlockSpec((DM, DFF), lambda i: (0, 0)),
            pl.BlockSpec((1, DFF), lambda i: (0, 0)),
            pl.BlockSpec((DFF, DM), lambda i: (0, 0)),
            pl.BlockSpec((1, DM), lambda i: (0, 0)),
            pl.BlockSpec((1, DM), lambda i: (0, 0)),
            pl.BlockSpec((1, DM), lambda i: (0, 0)),
        ],
        out_specs=pl.BlockSpec((RT, DM), lambda i: (i, 0)),
        out_shape=jax.ShapeDtypeStruct((L, DM), F32),
    )(x2, ctx, Wo.astype(BF), row(bo), row(ln1_g), row(ln1_b),
      W1.astype(BF), row(b1), W2.astype(BF), row(b2), row(ln2_g),
      row(ln2_b))

    return out.reshape(1, L, DM)
